# baseline (speedup 1.0000x reference)
import functools

import jax
import jax.numpy as jnp
from jax import lax
from jax.experimental import pallas as pl
from jax.experimental.pallas import tpu as pltpu
from jax.experimental.pallas import tpu_sc as plsc

H = 64
NCORES = 2
NSUB = 16
LANES = 16
NBUF = 12
GG = 4
BLK_C = 64
BLK_G = 32


def _mesh():
    return plsc.VectorSubcoreMesh(
        core_axis_name="c", subcore_axis_name="s",
        num_cores=NCORES, num_subcores=NSUB)


def _pad_edges(ei, blk):
    src, dst = ei[0], ei[1]
    e = src.shape[0]
    unit = NSUB * blk
    e_pad = -(-e // unit) * unit
    pad = e_pad - e
    src = jnp.concatenate([src, jnp.zeros((pad,), jnp.int32)])
    dst = jnp.concatenate([dst, jnp.full((pad,), -1, jnp.int32)])
    return src.reshape(e_pad // blk, blk), dst.reshape(e_pad // blk, blk), e_pad


def _zero_fill(buf, nrows, ngrp):
    z = jnp.zeros((LANES,), jnp.float32)

    def body(j, _):
        buf[j // ngrp, pl.ds((j % ngrp) * LANES, LANES)] = z
        return 0

    lax.fori_loop(0, nrows * ngrp, body, 0)


def _chunks(total, step):
    out, off = [], 0
    while off < total:
        n = min(step, total - off)
        out.append((off, n))
        off += n
    return out


def _remap_block(src_stage, dst_stage, i, src_work, dst_work,
                 base, r_half, blk):
    lane = lax.iota(jnp.int32, LANES)
    for j in range(blk // LANES):
        off = j * LANES
        dummy = r_half + ((i * (blk // LANES) + j) % 4) * LANES + lane
        d = dst_stage[i, pl.ds(off, LANES)]
        owned = (d >= base) & (d < base + r_half)
        dst_work[pl.ds(off, LANES)] = jnp.where(owned, d - base, dummy)
        if src_stage is not None:
            sv = src_stage[i, pl.ds(off, LANES)]
            src_work[pl.ds(off, LANES)] = jnp.where(owned, sv, 0)


def _acc_zero_and_barrier(accs, s):
    for acc, zbuf in accs:
        zrows = acc.shape[0] // NSUB
        znr = zbuf.shape[0]
        for off, n in _chunks(zrows, znr):
            pltpu.sync_copy(zbuf.at[pl.ds(0, n)],
                            acc.at[pl.ds(s * zrows + off, n)])
    plsc.subcore_barrier()


def _acc_write_out(accs_outs, s, c):
    plsc.subcore_barrier()
    for acc, out in accs_outs:
        zrows = acc.shape[0] // NSUB
        for off, n in _chunks(zrows, 256):
            pltpu.sync_copy(acc.at[pl.ds(s * zrows + off, n)],
                            out.at[c, pl.ds(s * zrows + off, n)])


@functools.lru_cache(maxsize=None)
def _make_agg(specs, r_half, r_acc):
    n_types = len(specs)
    blks = sorted({blk for _, blk in specs})

    def body(*refs):
        ins = refs[:3 * n_types]
        outs = refs[3 * n_types:4 * n_types]
        accs = refs[4 * n_types:5 * n_types]
        rest = list(refs[5 * n_types:])
        stages = {}
        works = {}
        rows = {}
        for blk in blks:
            stages[blk] = (rest.pop(0), rest.pop(0))
            works[blk] = (rest[:NBUF], rest[NBUF:2 * NBUF])
            del rest[:2 * NBUF]
            rows[blk] = rest[:NBUF]
            del rest[:NBUF]
        sems = rest[:NBUF]
        c = lax.axis_index("c")
        s = lax.axis_index("s")
        base = c * r_half

        zbuf = rows[blks[-1]][0]
        _zero_fill(zbuf, zbuf.shape[0], 4)
        _acc_zero_and_barrier([(acc, zbuf) for acc in accs], s)

        for t, (e_pad, blk) in enumerate(specs):
            src_hbm, dst_hbm, table = ins[3 * t:3 * t + 3]
            acc = accs[t]
            src_stage, dst_stage = stages[blk]
            src_work, dst_work = works[blk]
            row_bufs = rows[blk]
            nblk = e_pad // (NSUB * blk)
            sg = GG * NBUF
            nsg, tail = divmod(nblk, sg)

            def do_blocks(i0, nb, src_stage=src_stage, dst_stage=dst_stage,
                          table=table, acc=acc, src_work=src_work,
                          dst_work=dst_work, row_bufs=row_bufs, blk=blk):
                gh = []
                for i in range(nb):
                    _remap_block(src_stage, dst_stage, i0 + i,
                                 src_work[i], dst_work[i], base, r_half, blk)
                    gh.append(pltpu.async_copy(
                        table.at[src_work[i]], row_bufs[i], sems[i]))
                for i in range(nb):
                    gh[i].wait()

            def sg_body(q, _, src_hbm=src_hbm, dst_hbm=dst_hbm,
                        src_stage=src_stage, dst_stage=dst_stage,
                        do_blocks=do_blocks, nblk=nblk, sg=sg):
                row0 = s * nblk + q * sg
                pltpu.sync_copy(src_hbm.at[pl.ds(row0, sg)], src_stage)
                pltpu.sync_copy(dst_hbm.at[pl.ds(row0, sg)], dst_stage)
                for gg in range(GG):
                    do_blocks(gg * NBUF, NBUF)
                return 0

            lax.fori_loop(0, nsg, sg_body, 0)
            if tail:
                row0 = s * nblk + nsg * sg
                pltpu.sync_copy(src_hbm.at[pl.ds(row0, tail)],
                                src_stage.at[pl.ds(0, tail)])
                pltpu.sync_copy(dst_hbm.at[pl.ds(row0, tail)],
                                dst_stage.at[pl.ds(0, tail)])
                for i0, nb in _chunks(tail, NBUF):
                    do_blocks(i0, nb)

        _acc_write_out(list(zip(accs, outs)), s, c)

    out_type = [jax.ShapeDtypeStruct((NCORES, r_acc, H), jnp.float32)
                for _ in range(n_types)]
    scratch = [pltpu.VMEM_SHARED((r_acc, H), jnp.float32)
               for _ in range(n_types)]
    for blk in blks:
        scratch += [pltpu.VMEM((GG * NBUF, blk), jnp.int32),
                    pltpu.VMEM((GG * NBUF, blk), jnp.int32)]
        scratch += [pltpu.VMEM((blk,), jnp.int32) for _ in range(2 * NBUF)]
        scratch += [pltpu.VMEM((blk, H), jnp.float32) for _ in range(NBUF)]
    scratch += [pltpu.SemaphoreType.DMA for _ in range(NBUF)]
    return pl.kernel(body, out_type=out_type, mesh=_mesh(),
                     scratch_types=scratch,
                     compiler_params=pltpu.CompilerParams(
                         use_tc_tiling_on_sc=False))


@functools.lru_cache(maxsize=None)
def _make_counts(county_specs, case_specs, r_halves, r_accs):
    groups = (county_specs, case_specs)
    n_in = len(county_specs) + len(case_specs)
    blks = sorted({blk for g in groups for _, blk in g})

    def body(*refs):
        ins = refs[:n_in]
        outs = refs[n_in:n_in + 2]
        accs = refs[n_in + 2:n_in + 4]
        rest = list(refs[n_in + 4:])
        stages = {}
        works = {}
        ones = {}
        for blk in blks:
            stages[blk] = rest.pop(0)
            works[blk] = rest[:NBUF]
            del rest[:NBUF]
            ones[blk] = rest.pop(0)
        zbuf = rest.pop(0)
        sems = rest[:NBUF]
        c = lax.axis_index("c")
        s = lax.axis_index("s")

        _zero_fill(zbuf, 128, 1)
        _acc_zero_and_barrier([(acc, zbuf) for acc in accs], s)
        one = jnp.ones((LANES,), jnp.float32)
        for blk in blks:

            def fill_ones(j, _, ov=ones[blk]):
                ov[j, pl.ds(0, LANES)] = one
                return 0

            lax.fori_loop(0, blk, fill_ones, 0)

        k = 0
        for g in range(2):
            base = c * r_halves[g]
            r_half = r_halves[g]
            for e_pad, blk in groups[g]:
                dst_hbm = ins[k]
                k += 1
                acc = accs[g]
                dst_stage = stages[blk]
                dst_work = works[blk]
                ones_v = ones[blk]
                nblk = e_pad // (NSUB * blk)
                sg = GG * NBUF
                nsg, tail = divmod(nblk, sg)

                def do_blocks(i0, nb, dst_stage=dst_stage, acc=acc,
                              dst_work=dst_work, ones_v=ones_v,
                              base=base, r_half=r_half, blk=blk):
                    for i in range(nb):
                        _remap_block(None, dst_stage, i0 + i,
                                     None, dst_work[i], base, r_half, blk)
                    sh = []
                    for i in range(nb):
                        sh.append(pltpu.async_copy(
                            ones_v, acc.at[dst_work[i]], sems[i], add=True))
                    for i in range(nb):
                        sh[i].wait()

                def sg_body(q, _, dst_hbm=dst_hbm, dst_stage=dst_stage,
                            do_blocks=do_blocks, nblk=nblk, sg=sg):
                    row0 = s * nblk + q * sg
                    pltpu.sync_copy(dst_hbm.at[pl.ds(row0, sg)], dst_stage)
                    for gg in range(GG):
                        do_blocks(gg * NBUF, NBUF)
                    return 0

                lax.fori_loop(0, nsg, sg_body, 0)
                if tail:
                    row0 = s * nblk + nsg * sg
                    pltpu.sync_copy(dst_hbm.at[pl.ds(row0, tail)],
                                    dst_stage.at[pl.ds(0, tail)])
                    for i0, nb in _chunks(tail, NBUF):
                        do_blocks(i0, nb)

        _acc_write_out(list(zip(accs, outs)), s, c)

    out_type = [jax.ShapeDtypeStruct((NCORES, r_accs[0], LANES), jnp.float32),
                jax.ShapeDtypeStruct((NCORES, r_accs[1], LANES), jnp.float32)]
    scratch = [pltpu.VMEM_SHARED((r_accs[0], LANES), jnp.float32),
               pltpu.VMEM_SHARED((r_accs[1], LANES), jnp.float32)]
    for blk in blks:
        scratch += [pltpu.VMEM((GG * NBUF, blk), jnp.int32)]
        scratch += [pltpu.VMEM((blk,), jnp.int32) for _ in range(NBUF)]
        scratch += [pltpu.VMEM((blk, LANES), jnp.float32)]
    scratch += [pltpu.VMEM((128, LANES), jnp.float32)]
    scratch += [pltpu.SemaphoreType.DMA for _ in range(NBUF)]
    return pl.kernel(body, out_type=out_type, mesh=_mesh(),
                     scratch_types=scratch,
                     compiler_params=pltpu.CompilerParams(
                         use_tc_tiling_on_sc=False))


def _enc_body(x_ref, w_ref, b_ref, o_ref):
    o_ref[...] = jnp.dot(x_ref[...], w_ref[...],
                         preferred_element_type=jnp.float32) + b_ref[...]


def _encode(x, w, b, blk):
    n, f = x.shape
    return pl.pallas_call(
        _enc_body,
        grid=(n // blk,),
        in_specs=[pl.BlockSpec((blk, f), lambda i: (i, 0)),
                  pl.BlockSpec((f, H), lambda i: (0, 0)),
                  pl.BlockSpec((1, H), lambda i: (0, 0))],
        out_specs=pl.BlockSpec((blk, H), lambda i: (i, 0)),
        out_shape=jax.ShapeDtypeStruct((n, H), jnp.float32),
    )(x, w, b.reshape(1, H))


def _upd2_body(h_ref, a1_ref, a2_ref, cnt_ref, w1_ref, w2_ref, b_ref, o_ref):
    m = (jnp.dot(a1_ref[0], w1_ref[...], preferred_element_type=jnp.float32)
         + jnp.dot(a2_ref[0], w2_ref[...], preferred_element_type=jnp.float32))
    m = m / jnp.maximum(cnt_ref[0][:, 0:1], 1.0)
    o_ref[...] = h_ref[...] + jnp.maximum(m + b_ref[...], 0.0)


def _upd1_body(h_ref, a1_ref, cnt_ref, w1_ref, b_ref, o_ref):
    m = jnp.dot(a1_ref[0], w1_ref[...], preferred_element_type=jnp.float32)
    m = m / jnp.maximum(cnt_ref[0][:, 0:1], 1.0)
    o_ref[...] = h_ref[...] + jnp.maximum(m + b_ref[...], 0.0)


def _update(h, aggs, cnt_parts, ws, b, r_half, blk):
    n = h.shape[0]
    gc = r_half // blk
    agg_spec = pl.BlockSpec((1, blk, H), lambda i: (i // gc, i % gc, 0))
    cnt_spec = pl.BlockSpec((1, blk, LANES), lambda i: (i // gc, i % gc, 0))
    w_spec = pl.BlockSpec((H, H), lambda i: (0, 0))
    body = _upd2_body if len(aggs) == 2 else _upd1_body
    in_specs = ([pl.BlockSpec((blk, H), lambda i: (i, 0))]
                + [agg_spec] * len(aggs) + [cnt_spec]
                + [w_spec] * len(ws)
                + [pl.BlockSpec((1, H), lambda i: (0, 0))])
    return pl.pallas_call(
        body,
        grid=(n // blk,),
        in_specs=in_specs,
        out_specs=pl.BlockSpec((blk, H), lambda i: (i, 0)),
        out_shape=jax.ShapeDtypeStruct((n, H), jnp.float32),
    )(h, *aggs, cnt_parts, *ws, b.reshape(1, H))


NC = 10000
NCASE = 50000
RC = NC // NCORES
RS = NCASE // NCORES
RC_ACC = -(-(RC + 64) // 128) * 128
RS_ACC = -(-(RS + 64) // 128) * 128
NITER = 3


def kernel(x_county, x_case, ei_spatial, ei_genetic, ei_belongs,
           W_enc_county, b_enc_county, W_enc_case, b_enc_case,
           W_spatial, W_genetic, W_belongs, b_county, b_case):
    src_sp, dst_sp, ep_sp = _pad_edges(ei_spatial, BLK_C)
    src_ge, dst_ge, ep_ge = _pad_edges(ei_genetic, BLK_G)
    src_bl, dst_bl, ep_bl = _pad_edges(ei_belongs, BLK_C)

    hc = _encode(x_county, W_enc_county, b_enc_county, 2000)
    hs = _encode(x_case, W_enc_case, b_enc_case, 2000)

    cnt_c, cnt_s = _make_counts(((ep_sp, BLK_C), (ep_bl, BLK_C)),
                                ((ep_ge, BLK_G),),
                                (RC, RS), (RC_ACC, RS_ACC))(
        dst_sp, dst_bl, dst_ge)

    county_agg = _make_agg(((ep_sp, BLK_C), (ep_bl, BLK_C)), RC, RC_ACC)
    case_agg = _make_agg(((ep_ge, BLK_G),), RS, RS_ACC)

    for _ in range(NITER):
        agg_sp, agg_bl = county_agg(src_sp, dst_sp, hc, src_bl, dst_bl, hs)
        (agg_ge,) = case_agg(src_ge, dst_ge, hs)
        hc = _update(hc, [agg_sp, agg_bl], cnt_c,
                     [W_spatial, W_belongs], b_county, RC, 1000)
        hs = _update(hs, [agg_ge], cnt_s,
                     [W_genetic], b_case, RS, 1000)
    return hc, hs


# speedup vs baseline: 9.3894x; 9.3894x over previous
import functools

import jax
import jax.numpy as jnp
from jax import lax
from jax.experimental import pallas as pl
from jax.experimental.pallas import tpu as pltpu
from jax.experimental.pallas import tpu_sc as plsc

H = 64
NCORES = 2
NSUB = 16
NW = NCORES * NSUB
LANES = 16
NBUF = 5
GBLK = 32
CHUNK_V = 128
CHUNK_E = CHUNK_V * LANES
FLUSH_THR = 1024
CAP = FLUSH_THR + CHUNK_E + 2 * LANES


def _mesh():
    return plsc.VectorSubcoreMesh(
        core_axis_name="c", subcore_axis_name="s",
        num_cores=NCORES, num_subcores=NSUB)


def _pad_edges(ei):
    src, dst = ei[0], ei[1]
    e = src.shape[0]
    unit = 2 * CHUNK_E
    e_pad = -(-e // unit) * unit
    pad = e_pad - e
    src = jnp.concatenate([src, jnp.zeros((pad,), jnp.int32)])
    dst = jnp.concatenate([dst, jnp.full((pad,), -1, jnp.int32)])
    return (src.reshape(e_pad // LANES, LANES),
            dst.reshape(e_pad // LANES, LANES), e_pad)


def _zero2d(buf):
    z = jnp.zeros((LANES,), jnp.float32)
    ng = buf.shape[1] // LANES

    def body(j, _):
        buf[j // ng, pl.ds((j % ng) * LANES, LANES)] = z
        return 0

    lax.fori_loop(0, buf.shape[0] * ng, body, 0)


def _zero1d(buf):
    z = jnp.zeros((LANES,), jnp.float32)

    def body(j, _):
        buf[pl.ds(j * LANES, LANES)] = z
        return 0

    lax.fori_loop(0, buf.shape[0] // LANES, body, 0)


def _scalar(x):
    if getattr(x, "ndim", 0) == 1:
        return lax.reduce_max(x, (0,))
    return x


@functools.lru_cache(maxsize=None)
def _make_agg(n_types, e_pads, r_tile, with_counts):
    acc_rows = r_tile + LANES

    def body(*refs):
        ins = refs[:3 * n_types]
        pos = 3 * n_types
        outs = refs[pos:pos + n_types]; pos += n_types
        if with_counts:
            cnt_out = refs[pos]; pos += 1
        accs = refs[pos:pos + n_types]; pos += n_types
        if with_counts:
            cnt_acc = refs[pos]; pos += 1
        srcA, dstA, srcB, dstB = refs[pos:pos + 4]; pos += 4
        csrc, cdst = refs[pos:pos + 2]; pos += 2
        rows = refs[pos:pos + NBUF]; pos += NBUF
        semA, semB = refs[pos:pos + 2]; pos += 2
        sems = refs[pos:pos + NBUF]; pos += NBUF
        c = lax.axis_index("c")
        s = lax.axis_index("s")
        w = s * NCORES + c
        lo = w * r_tile
        hi = lo + r_tile
        ones_f = jnp.ones((LANES,), jnp.float32)
        colv = [lax.iota(jnp.int32, LANES) + j * LANES
                for j in range(H // LANES)]

        for a in accs:
            _zero2d(a)
        if with_counts:
            _zero1d(cnt_acc)

        for t in range(n_types):
            src_hbm, dst_hbm, table = ins[3 * t:3 * t + 3]
            acc = accs[t]
            nch = e_pads[t] // CHUNK_E

            def issue(ch, stg_s, stg_d, sem, src_hbm=src_hbm,
                      dst_hbm=dst_hbm):
                pltpu.async_copy(
                    src_hbm.at[pl.ds(ch * CHUNK_V, CHUNK_V)], stg_s, sem)
                pltpu.async_copy(
                    dst_hbm.at[pl.ds(ch * CHUNK_V, CHUNK_V)], stg_d, sem)

            def wait_stage(ch, stg_s, stg_d, sem, src_hbm=src_hbm,
                           dst_hbm=dst_hbm):
                pltpu.make_async_copy(
                    src_hbm.at[pl.ds(ch * CHUNK_V, CHUNK_V)],
                    stg_s, sem).wait()
                pltpu.make_async_copy(
                    dst_hbm.at[pl.ds(ch * CHUNK_V, CHUNK_V)],
                    stg_d, sem).wait()

            def scan_chunk(stg_s, stg_d, cur):
                def scan_v(v, cur):
                    d = stg_d[v, pl.ds(0, LANES)]
                    sv = stg_s[v, pl.ds(0, LANES)]
                    owned = (d >= lo) & (d < hi)
                    oi = owned.astype(jnp.int32)
                    pos = cur + plsc.cumsum(oi) - oi
                    plsc.store_scatter(cdst, [pos], d - lo, mask=owned)
                    plsc.store_scatter(csrc, [pos], sv, mask=owned)
                    if with_counts:
                        li = jnp.where(owned, d - lo, r_tile)
                        plsc.addupdate_scatter(cnt_acc, [li], ones_f)
                    return cur + _scalar(
                        plsc.all_reduce_population_count(owned))

                return lax.fori_loop(0, CHUNK_V, scan_v, cur)

            def flush(cur, final, acc=acc, table=table):
                if final:
                    dmy = jnp.zeros((LANES,), jnp.int32) + r_tile
                    zz = jnp.zeros((LANES,), jnp.int32)
                    cdst[pl.ds(cur, LANES)] = dmy
                    cdst[pl.ds(cur + LANES, LANES)] = dmy
                    csrc[pl.ds(cur, LANES)] = zz
                    csrc[pl.ds(cur + LANES, LANES)] = zz
                    nb = (cur + GBLK - 1) >> 5
                else:
                    nb = jnp.where(cur >= FLUSH_THR, cur >> 5, 0)
                ngr = (nb + NBUF - 1) // NBUF

                def grp(gi, _):
                    for i in range(NBUF):
                        bi = gi * NBUF + i

                        @pl.when(bi < nb)
                        def _(i=i, bi=bi):
                            pltpu.async_copy(
                                table.at[csrc.at[pl.ds(bi * GBLK, GBLK)]],
                                rows[i], sems[i])
                    for i in range(NBUF):
                        bi = gi * NBUF + i

                        @pl.when(bi < nb)
                        def _(i=i, bi=bi):
                            pltpu.make_async_copy(
                                table.at[csrc.at[pl.ds(bi * GBLK, GBLK)]],
                                rows[i], sems[i]).wait()

                            def edge_body(e, _, i=i, bi=bi, acc=acc):
                                eidx = (jnp.zeros((LANES,), jnp.int32)
                                        + (bi * GBLK + e))
                                dstl = plsc.load_gather(cdst, [eidx])
                                for j in range(H // LANES):
                                    val = rows[i][e, pl.ds(j * LANES, LANES)]
                                    plsc.addupdate_scatter(
                                        acc, [dstl, colv[j]], val)
                                return 0

                            lax.fori_loop(0, GBLK, edge_body, 0)
                    return 0

                lax.fori_loop(0, ngr, grp, 0)
                if final:
                    return jnp.int32(0)
                rem_off = nb * GBLK
                for vv in range(2):
                    td = cdst[pl.ds(rem_off + vv * LANES, LANES)]
                    ts = csrc[pl.ds(rem_off + vv * LANES, LANES)]
                    cdst[pl.ds(vv * LANES, LANES)] = td
                    csrc[pl.ds(vv * LANES, LANES)] = ts
                return cur - (nb << 5)

            issue(0, srcA, dstA, semA)

            def pair_body(p, cur, issue=issue, wait_stage=wait_stage,
                          scan_chunk=scan_chunk, flush=flush, nch=nch):
                ch0 = 2 * p
                wait_stage(ch0, srcA, dstA, semA)
                issue(ch0 + 1, srcB, dstB, semB)
                cur = scan_chunk(srcA, dstA, cur)
                cur = flush(cur, False)
                wait_stage(ch0 + 1, srcB, dstB, semB)

                @pl.when(ch0 + 2 < nch)
                def _():
                    issue(ch0 + 2, srcA, dstA, semA)

                cur = scan_chunk(srcB, dstB, cur)
                cur = flush(cur, False)
                return cur

            cur = lax.fori_loop(0, nch // 2, pair_body, jnp.int32(0))
            flush(cur, True)

        for t in range(n_types):
            pltpu.sync_copy(accs[t].at[pl.ds(0, r_tile)], outs[t].at[w])
        if with_counts:
            pltpu.sync_copy(cnt_acc.at[pl.ds(0, r_tile)], cnt_out.at[w])

    out_type = [jax.ShapeDtypeStruct((NW, r_tile, H), jnp.float32)
                for _ in range(n_types)]
    if with_counts:
        out_type.append(jax.ShapeDtypeStruct((NW, r_tile), jnp.float32))
    scratch = [pltpu.VMEM((acc_rows, H), jnp.float32)
               for _ in range(n_types)]
    if with_counts:
        scratch.append(pltpu.VMEM((acc_rows,), jnp.float32))
    scratch += [pltpu.VMEM((CHUNK_V, LANES), jnp.int32) for _ in range(4)]
    scratch += [pltpu.VMEM((CAP,), jnp.int32) for _ in range(2)]
    scratch += [pltpu.VMEM((GBLK, H), jnp.float32) for _ in range(NBUF)]
    scratch += [pltpu.SemaphoreType.DMA for _ in range(2 + NBUF)]
    return pl.kernel(body, out_type=out_type, mesh=_mesh(),
                     scratch_types=scratch,
                     compiler_params=pltpu.CompilerParams(
                         use_tc_tiling_on_sc=False,
                         needs_layout_passes=False))


def _enc_body(x_ref, w_ref, b_ref, o_ref):
    o_ref[...] = jnp.dot(x_ref[...], w_ref[...],
                         preferred_element_type=jnp.float32) + b_ref[...]


def _encode(x, w, b, blk):
    n, f = x.shape
    return pl.pallas_call(
        _enc_body,
        grid=(n // blk,),
        in_specs=[pl.BlockSpec((blk, f), lambda i: (i, 0)),
                  pl.BlockSpec((f, H), lambda i: (0, 0)),
                  pl.BlockSpec((1, H), lambda i: (0, 0))],
        out_specs=pl.BlockSpec((blk, H), lambda i: (i, 0)),
        out_shape=jax.ShapeDtypeStruct((n, H), jnp.float32),
    )(x, w, b.reshape(1, H))


def _upd2_body(h_ref, a1_ref, a2_ref, cnt_ref, w1_ref, w2_ref, b_ref, o_ref):
    m = (jnp.dot(a1_ref[...], w1_ref[...], preferred_element_type=jnp.float32)
         + jnp.dot(a2_ref[...], w2_ref[...],
                   preferred_element_type=jnp.float32))
    m = m / jnp.maximum(cnt_ref[...], 1.0)
    o_ref[...] = h_ref[...] + jnp.maximum(m + b_ref[...], 0.0)


def _upd1_body(h_ref, a1_ref, cnt_ref, w1_ref, b_ref, o_ref):
    m = jnp.dot(a1_ref[...], w1_ref[...], preferred_element_type=jnp.float32)
    m = m / jnp.maximum(cnt_ref[...], 1.0)
    o_ref[...] = h_ref[...] + jnp.maximum(m + b_ref[...], 0.0)


def _update(h, aggs, cnt, ws, b, blk):
    n = h.shape[0]
    hspec = pl.BlockSpec((blk, H), lambda i: (i, 0))
    cnt_spec = pl.BlockSpec((blk, 1), lambda i: (i, 0))
    w_spec = pl.BlockSpec((H, H), lambda i: (0, 0))
    body = _upd2_body if len(aggs) == 2 else _upd1_body
    in_specs = ([hspec] + [hspec] * len(aggs) + [cnt_spec]
                + [w_spec] * len(ws)
                + [pl.BlockSpec((1, H), lambda i: (0, 0))])
    return pl.pallas_call(
        body,
        grid=(n // blk,),
        in_specs=in_specs,
        out_specs=hspec,
        out_shape=jax.ShapeDtypeStruct((n, H), jnp.float32),
    )(h, *aggs, cnt, *ws, b.reshape(1, H))


NC = 10000
NCASE = 50000
RT_C = 320
RT_S = 1568
NPC = NW * RT_C
NPS = NW * RT_S
NITER = 3


def kernel(x_county, x_case, ei_spatial, ei_genetic, ei_belongs,
           W_enc_county, b_enc_county, W_enc_case, b_enc_case,
           W_spatial, W_genetic, W_belongs, b_county, b_case):
    src_sp, dst_sp, ep_sp = _pad_edges(ei_spatial)
    src_ge, dst_ge, ep_ge = _pad_edges(ei_genetic)
    src_bl, dst_bl, ep_bl = _pad_edges(ei_belongs)

    xc = jnp.pad(x_county, ((0, NPC - NC), (0, 0)))
    xs = jnp.pad(x_case, ((0, NPS - NCASE), (0, 0)))
    hc = _encode(xc, W_enc_county, b_enc_county, 1024)
    hs = _encode(xs, W_enc_case, b_enc_case, 1024)

    county1 = _make_agg(2, (ep_sp, ep_bl), RT_C, True)
    county2 = _make_agg(2, (ep_sp, ep_bl), RT_C, False)
    case1 = _make_agg(1, (ep_ge,), RT_S, True)
    case2 = _make_agg(1, (ep_ge,), RT_S, False)

    cnt_c = cnt_s = None
    for it in range(NITER):
        if it == 0:
            agg_sp, agg_bl, cnt_c = county1(src_sp, dst_sp, hc,
                                            src_bl, dst_bl, hs)
            agg_ge, cnt_s = case1(src_ge, dst_ge, hs)
            cnt_c = cnt_c.reshape(NPC, 1)
            cnt_s = cnt_s.reshape(NPS, 1)
        else:
            agg_sp, agg_bl = county2(src_sp, dst_sp, hc, src_bl, dst_bl, hs)
            (agg_ge,) = case2(src_ge, dst_ge, hs)
        hc = _update(hc, [agg_sp.reshape(NPC, H), agg_bl.reshape(NPC, H)],
                     cnt_c, [W_spatial, W_belongs], b_county, 1024)
        hs = _update(hs, [agg_ge.reshape(NPS, H)],
                     cnt_s, [W_genetic], b_case, 1024)
    return hc[:NC], hs[:NCASE]


# speedup vs baseline: 11.3688x; 1.2108x over previous
import functools

import jax
import jax.numpy as jnp
from jax import lax
from jax.experimental import pallas as pl
from jax.experimental.pallas import tpu as pltpu
from jax.experimental.pallas import tpu_sc as plsc

H = 64
NCORES = 2
NSUB = 16
NW = NCORES * NSUB
LANES = 16
NBUF = 5
GBLK = 32
CHUNK_V = 128
CHUNK_E = CHUNK_V * LANES
FLUSH_THR = 1024
CAP = FLUSH_THR + CHUNK_E + 2 * LANES


def _mesh():
    return plsc.VectorSubcoreMesh(
        core_axis_name="c", subcore_axis_name="s",
        num_cores=NCORES, num_subcores=NSUB)


def _pad_edges(ei):
    src, dst = ei[0], ei[1]
    e = src.shape[0]
    unit = 2 * CHUNK_E
    e_pad = -(-e // unit) * unit
    pad = e_pad - e
    src = jnp.concatenate([src, jnp.zeros((pad,), jnp.int32)])
    dst = jnp.concatenate([dst, jnp.full((pad,), -1, jnp.int32)])
    return (src.reshape(e_pad // LANES, LANES),
            dst.reshape(e_pad // LANES, LANES), e_pad)


def _zero2d(buf):
    z = jnp.zeros((LANES,), jnp.float32)
    ng = buf.shape[1] // LANES

    def body(j, _):
        buf[j // ng, pl.ds((j % ng) * LANES, LANES)] = z
        return 0

    lax.fori_loop(0, buf.shape[0] * ng, body, 0)


def _zero1d(buf):
    z = jnp.zeros((LANES,), jnp.float32)

    def body(j, _):
        buf[pl.ds(j * LANES, LANES)] = z
        return 0

    lax.fori_loop(0, buf.shape[0] // LANES, body, 0)


def _scalar(x):
    if getattr(x, "ndim", 0) == 1:
        return lax.reduce_max(x, (0,))
    return x


@functools.lru_cache(maxsize=None)
def _make_agg(n_types, e_pads, r_tile, with_counts):
    acc_rows = r_tile + LANES

    def body(*refs):
        ins = refs[:3 * n_types]
        pos = 3 * n_types
        outs = refs[pos:pos + n_types]; pos += n_types
        if with_counts:
            cnt_out = refs[pos]; pos += 1
        accs = refs[pos:pos + n_types]; pos += n_types
        if with_counts:
            cnt_acc = refs[pos]; pos += 1
        srcA, dstA, srcB, dstB = refs[pos:pos + 4]; pos += 4
        csrc, cdst = refs[pos:pos + 2]; pos += 2
        rows = refs[pos:pos + NBUF]; pos += NBUF
        semA, semB = refs[pos:pos + 2]; pos += 2
        sems = refs[pos:pos + NBUF]; pos += NBUF
        c = lax.axis_index("c")
        s = lax.axis_index("s")
        w = s * NCORES + c
        lo = w * r_tile
        hi = lo + r_tile
        ones_f = jnp.ones((LANES,), jnp.float32)
        colv = [lax.iota(jnp.int32, LANES) + j * LANES
                for j in range(H // LANES)]

        for a in accs:
            _zero2d(a)
        if with_counts:
            _zero1d(cnt_acc)

        for t in range(n_types):
            src_hbm, dst_hbm, table = ins[3 * t:3 * t + 3]
            acc = accs[t]
            nch = e_pads[t] // CHUNK_E

            def issue(ch, stg_s, stg_d, sem, src_hbm=src_hbm,
                      dst_hbm=dst_hbm):
                pltpu.async_copy(
                    src_hbm.at[pl.ds(ch * CHUNK_V, CHUNK_V)], stg_s, sem)
                pltpu.async_copy(
                    dst_hbm.at[pl.ds(ch * CHUNK_V, CHUNK_V)], stg_d, sem)

            def wait_stage(ch, stg_s, stg_d, sem, src_hbm=src_hbm,
                           dst_hbm=dst_hbm):
                pltpu.make_async_copy(
                    src_hbm.at[pl.ds(ch * CHUNK_V, CHUNK_V)],
                    stg_s, sem).wait()
                pltpu.make_async_copy(
                    dst_hbm.at[pl.ds(ch * CHUNK_V, CHUNK_V)],
                    stg_d, sem).wait()

            def scan_chunk(stg_s, stg_d, cur):
                def scan_v(v, curv):
                    d = stg_d[v, pl.ds(0, LANES)]
                    sv = stg_s[v, pl.ds(0, LANES)]
                    owned = (d >= lo) & (d < hi)
                    oi = owned.astype(jnp.int32)
                    pos = curv + plsc.cumsum(oi) - oi
                    plsc.store_scatter(cdst, [pos], d - lo, mask=owned)
                    plsc.store_scatter(csrc, [pos], sv, mask=owned)
                    n = plsc.all_reduce_population_count(owned)
                    if getattr(n, "ndim", 0) == 0:
                        n = jnp.zeros((LANES,), jnp.int32) + n
                    return curv + n

                curv = jnp.zeros((LANES,), jnp.int32) + cur
                curv = lax.fori_loop(0, CHUNK_V, scan_v, curv)
                return _scalar(curv)

            def flush(cur, final, acc=acc, table=table):
                if final:
                    dmy = jnp.zeros((LANES,), jnp.int32) + r_tile
                    zz = jnp.zeros((LANES,), jnp.int32)
                    cdst[pl.ds(cur, LANES)] = dmy
                    cdst[pl.ds(cur + LANES, LANES)] = dmy
                    csrc[pl.ds(cur, LANES)] = zz
                    csrc[pl.ds(cur + LANES, LANES)] = zz
                    nb = (cur + GBLK - 1) >> 5
                else:
                    nb = jnp.where(cur >= FLUSH_THR, cur >> 5, 0)
                ngr = (nb + NBUF - 1) // NBUF

                def grp(gi, _):
                    for i in range(NBUF):
                        bi = gi * NBUF + i

                        @pl.when(bi < nb)
                        def _(i=i, bi=bi):
                            pltpu.async_copy(
                                table.at[csrc.at[pl.ds(bi * GBLK, GBLK)]],
                                rows[i], sems[i])
                    for i in range(NBUF):
                        bi = gi * NBUF + i

                        @pl.when(bi < nb)
                        def _(i=i, bi=bi):
                            pltpu.make_async_copy(
                                table.at[csrc.at[pl.ds(bi * GBLK, GBLK)]],
                                rows[i], sems[i]).wait()

                            if with_counts:
                                for v in range(GBLK // LANES):
                                    dv = cdst[pl.ds(bi * GBLK + v * LANES,
                                                    LANES)]
                                    plsc.addupdate_scatter(
                                        cnt_acc, [dv], ones_f)

                            def edge_body(e, _, i=i, bi=bi, acc=acc):
                                eidx = (jnp.zeros((LANES,), jnp.int32)
                                        + (bi * GBLK + e))
                                dstl = plsc.load_gather(cdst, [eidx])
                                for j in range(H // LANES):
                                    val = rows[i][e, pl.ds(j * LANES, LANES)]
                                    plsc.addupdate_scatter(
                                        acc, [dstl, colv[j]], val)
                                return 0

                            lax.fori_loop(0, GBLK, edge_body, 0)
                    return 0

                lax.fori_loop(0, ngr, grp, 0)
                if final:
                    return jnp.int32(0)
                rem_off = nb * GBLK
                for vv in range(2):
                    td = cdst[pl.ds(rem_off + vv * LANES, LANES)]
                    ts = csrc[pl.ds(rem_off + vv * LANES, LANES)]
                    cdst[pl.ds(vv * LANES, LANES)] = td
                    csrc[pl.ds(vv * LANES, LANES)] = ts
                return cur - (nb << 5)

            issue(0, srcA, dstA, semA)

            def pair_body(p, cur, issue=issue, wait_stage=wait_stage,
                          scan_chunk=scan_chunk, flush=flush, nch=nch):
                ch0 = 2 * p
                wait_stage(ch0, srcA, dstA, semA)
                issue(ch0 + 1, srcB, dstB, semB)
                cur = scan_chunk(srcA, dstA, cur)
                cur = flush(cur, False)
                wait_stage(ch0 + 1, srcB, dstB, semB)

                @pl.when(ch0 + 2 < nch)
                def _():
                    issue(ch0 + 2, srcA, dstA, semA)

                cur = scan_chunk(srcB, dstB, cur)
                cur = flush(cur, False)
                return cur

            cur = lax.fori_loop(0, nch // 2, pair_body, jnp.int32(0))
            flush(cur, True)

        for t in range(n_types):
            pltpu.sync_copy(accs[t].at[pl.ds(0, r_tile)], outs[t].at[w])
        if with_counts:
            pltpu.sync_copy(cnt_acc.at[pl.ds(0, r_tile)], cnt_out.at[w])

    out_type = [jax.ShapeDtypeStruct((NW, r_tile, H), jnp.float32)
                for _ in range(n_types)]
    if with_counts:
        out_type.append(jax.ShapeDtypeStruct((NW, r_tile), jnp.float32))
    scratch = [pltpu.VMEM((acc_rows, H), jnp.float32)
               for _ in range(n_types)]
    if with_counts:
        scratch.append(pltpu.VMEM((acc_rows,), jnp.float32))
    scratch += [pltpu.VMEM((CHUNK_V, LANES), jnp.int32) for _ in range(4)]
    scratch += [pltpu.VMEM((CAP,), jnp.int32) for _ in range(2)]
    scratch += [pltpu.VMEM((GBLK, H), jnp.float32) for _ in range(NBUF)]
    scratch += [pltpu.SemaphoreType.DMA for _ in range(2 + NBUF)]
    return pl.kernel(body, out_type=out_type, mesh=_mesh(),
                     scratch_types=scratch,
                     compiler_params=pltpu.CompilerParams(
                         use_tc_tiling_on_sc=False,
                         needs_layout_passes=False))


def _enc_body(x_ref, w_ref, b_ref, o_ref):
    o_ref[...] = jnp.dot(x_ref[...], w_ref[...],
                         preferred_element_type=jnp.float32) + b_ref[...]


def _encode(x, w, b, blk):
    n, f = x.shape
    return pl.pallas_call(
        _enc_body,
        grid=(n // blk,),
        in_specs=[pl.BlockSpec((blk, f), lambda i: (i, 0)),
                  pl.BlockSpec((f, H), lambda i: (0, 0)),
                  pl.BlockSpec((1, H), lambda i: (0, 0))],
        out_specs=pl.BlockSpec((blk, H), lambda i: (i, 0)),
        out_shape=jax.ShapeDtypeStruct((n, H), jnp.float32),
    )(x, w, b.reshape(1, H))


def _upd2_body(h_ref, a1_ref, a2_ref, cnt_ref, w1_ref, w2_ref, b_ref, o_ref):
    m = (jnp.dot(a1_ref[...], w1_ref[...], preferred_element_type=jnp.float32)
         + jnp.dot(a2_ref[...], w2_ref[...],
                   preferred_element_type=jnp.float32))
    m = m / jnp.maximum(cnt_ref[...], 1.0)
    o_ref[...] = h_ref[...] + jnp.maximum(m + b_ref[...], 0.0)


def _upd1_body(h_ref, a1_ref, cnt_ref, w1_ref, b_ref, o_ref):
    m = jnp.dot(a1_ref[...], w1_ref[...], preferred_element_type=jnp.float32)
    m = m / jnp.maximum(cnt_ref[...], 1.0)
    o_ref[...] = h_ref[...] + jnp.maximum(m + b_ref[...], 0.0)


def _update(h, aggs, cnt, ws, b, blk):
    n = h.shape[0]
    hspec = pl.BlockSpec((blk, H), lambda i: (i, 0))
    cnt_spec = pl.BlockSpec((blk, 1), lambda i: (i, 0))
    w_spec = pl.BlockSpec((H, H), lambda i: (0, 0))
    body = _upd2_body if len(aggs) == 2 else _upd1_body
    in_specs = ([hspec] + [hspec] * len(aggs) + [cnt_spec]
                + [w_spec] * len(ws)
                + [pl.BlockSpec((1, H), lambda i: (0, 0))])
    return pl.pallas_call(
        body,
        grid=(n // blk,),
        in_specs=in_specs,
        out_specs=hspec,
        out_shape=jax.ShapeDtypeStruct((n, H), jnp.float32),
    )(h, *aggs, cnt, *ws, b.reshape(1, H))


NC = 10000
NCASE = 50000
RT_C = 320
RT_S = 1568
NPC = NW * RT_C
NPS = NW * RT_S
NITER = 3


def kernel(x_county, x_case, ei_spatial, ei_genetic, ei_belongs,
           W_enc_county, b_enc_county, W_enc_case, b_enc_case,
           W_spatial, W_genetic, W_belongs, b_county, b_case):
    src_sp, dst_sp, ep_sp = _pad_edges(ei_spatial)
    src_ge, dst_ge, ep_ge = _pad_edges(ei_genetic)
    src_bl, dst_bl, ep_bl = _pad_edges(ei_belongs)

    xc = jnp.pad(x_county, ((0, NPC - NC), (0, 0)))
    xs = jnp.pad(x_case, ((0, NPS - NCASE), (0, 0)))
    hc = _encode(xc, W_enc_county, b_enc_county, 1024)
    hs = _encode(xs, W_enc_case, b_enc_case, 1024)

    county1 = _make_agg(2, (ep_sp, ep_bl), RT_C, True)
    county2 = _make_agg(2, (ep_sp, ep_bl), RT_C, False)
    case1 = _make_agg(1, (ep_ge,), RT_S, True)
    case2 = _make_agg(1, (ep_ge,), RT_S, False)

    cnt_c = cnt_s = None
    for it in range(NITER):
        if it == 0:
            agg_sp, agg_bl, cnt_c = county1(src_sp, dst_sp, hc,
                                            src_bl, dst_bl, hs)
            agg_ge, cnt_s = case1(src_ge, dst_ge, hs)
            cnt_c = cnt_c.reshape(NPC, 1)
            cnt_s = cnt_s.reshape(NPS, 1)
        else:
            agg_sp, agg_bl = county2(src_sp, dst_sp, hc, src_bl, dst_bl, hs)
            (agg_ge,) = case2(src_ge, dst_ge, hs)
        hc = _update(hc, [agg_sp.reshape(NPC, H), agg_bl.reshape(NPC, H)],
                     cnt_c, [W_spatial, W_belongs], b_county, 1024)
        hs = _update(hs, [agg_ge.reshape(NPS, H)],
                     cnt_s, [W_genetic], b_case, 1024)
    return hc[:NC], hs[:NCASE]


# speedup vs baseline: 15.9486x; 1.4028x over previous
import functools

import jax
import jax.numpy as jnp
from jax import lax
from jax.experimental import pallas as pl
from jax.experimental.pallas import tpu as pltpu
from jax.experimental.pallas import tpu_sc as plsc

H = 64
NCORES = 2
NSUB = 16
NW = NCORES * NSUB
LANES = 16
NBUF = 5
GBLK = 32
CHUNK_V = 128
CHUNK_E = CHUNK_V * LANES
FLUSH_THR = 1024
CAP = FLUSH_THR + CHUNK_E + 2 * LANES


def _mesh():
    return plsc.VectorSubcoreMesh(
        core_axis_name="c", subcore_axis_name="s",
        num_cores=NCORES, num_subcores=NSUB)


def _pad_edges(ei):
    src, dst = ei[0], ei[1]
    e = src.shape[0]
    unit = 2 * CHUNK_E
    e_pad = -(-e // unit) * unit
    pad = e_pad - e
    src = jnp.concatenate([src, jnp.zeros((pad,), jnp.int32)])
    dst = jnp.concatenate([dst, jnp.full((pad,), -1, jnp.int32)])
    return (src.reshape(e_pad // LANES, LANES),
            dst.reshape(e_pad // LANES, LANES), e_pad)


def _zero2d(buf):
    z = jnp.zeros((LANES,), jnp.float32)
    ng = buf.shape[1] // LANES

    def body(j, _):
        buf[j // ng, pl.ds((j % ng) * LANES, LANES)] = z
        return 0

    lax.fori_loop(0, buf.shape[0] * ng, body, 0)


def _zero1d(buf):
    z = jnp.zeros((LANES,), jnp.float32)

    def body(j, _):
        buf[pl.ds(j * LANES, LANES)] = z
        return 0

    lax.fori_loop(0, buf.shape[0] // LANES, body, 0)


def _scalar(x):
    if getattr(x, "ndim", 0) == 1:
        return lax.reduce_max(x, (0,))
    return x


@functools.lru_cache(maxsize=None)
def _make_agg(n_types, e_pads, r_tile, mode):
    acc_rows = r_tile + LANES
    first = mode == "first"
    cols = [ep + CAP + CHUNK_E for ep in e_pads]
    n_in = 3 * n_types if first else 4 * n_types

    def body(*refs):
        ins = refs[:n_in]
        pos = n_in
        outs = refs[pos:pos + n_types]; pos += n_types
        if first:
            cnt_out = refs[pos]; pos += 1
            comp_outs = refs[pos:pos + 3 * n_types]; pos += 3 * n_types
        accs = refs[pos:pos + n_types]; pos += n_types
        if first:
            cnt_acc = refs[pos]; pos += 1
            srcA, dstA, srcB, dstB = refs[pos:pos + 4]; pos += 4
            dmybuf = refs[pos]; pos += 1
        else:
            srcA, dstA = refs[pos:pos + 2]; pos += 2
        mbuf = refs[pos]; pos += 1
        csrc, cdst = refs[pos:pos + 2]; pos += 2
        rows = refs[pos:pos + NBUF]; pos += NBUF
        semA, semB = refs[pos:pos + 2]; pos += 2
        sems = refs[pos:pos + NBUF]; pos += NBUF
        c = lax.axis_index("c")
        s = lax.axis_index("s")
        w = s * NCORES + c
        lo = w * r_tile
        hi = lo + r_tile
        ones_f = jnp.ones((LANES,), jnp.float32)
        colv = [lax.iota(jnp.int32, LANES) + j * LANES
                for j in range(H // LANES)]

        for a in accs:
            _zero2d(a)
        if first:
            _zero1d(cnt_acc)
            mone = jnp.zeros((LANES,), jnp.int32) - 1

            def fill_dmy(j, _):
                dmybuf[pl.ds(j * LANES, LANES)] = mone
                return 0

            lax.fori_loop(0, CHUNK_E // LANES, fill_dmy, 0)

        for t in range(n_types):
            if first:
                src_hbm, dst_hbm, table = ins[3 * t:3 * t + 3]
                comp_src, comp_dst, mcnt_out = comp_outs[3 * t:3 * t + 3]
            else:
                comp_src, comp_dst, mcnt_hbm, table = ins[4 * t:4 * t + 4]
            acc = accs[t]

            def scan_v_factory(read_d, read_s):
                def scan_v(v, curv):
                    d = read_d(v)
                    sv = read_s(v)
                    owned = (d >= lo) & (d < hi)
                    oi = owned.astype(jnp.int32)
                    p = curv + plsc.cumsum(oi) - oi
                    plsc.store_scatter(cdst, [p], d, mask=owned)
                    plsc.store_scatter(csrc, [p], sv, mask=owned)
                    n = plsc.all_reduce_population_count(owned)
                    if getattr(n, "ndim", 0) == 0:
                        n = jnp.zeros((LANES,), jnp.int32) + n
                    return curv + n
                return scan_v

            def flush(cur, hcur, final, acc=acc, table=table,
                      comp_src=(comp_src if first else None),
                      comp_dst=(comp_dst if first else None)):
                if final:
                    dmy = jnp.zeros((LANES,), jnp.int32) + (lo + r_tile)
                    zz = jnp.zeros((LANES,), jnp.int32)
                    cdst[pl.ds(cur, LANES)] = dmy
                    cdst[pl.ds(cur + LANES, LANES)] = dmy
                    csrc[pl.ds(cur, LANES)] = zz
                    csrc[pl.ds(cur + LANES, LANES)] = zz
                    nb = (cur + GBLK - 1) >> 5
                else:
                    nb = jnp.where(cur >= FLUSH_THR, cur >> 5, 0)
                ngr = (nb + NBUF - 1) // NBUF

                if first:
                    @pl.when(nb > 0)
                    def _():
                        pltpu.sync_copy(
                            csrc.at[pl.ds(0, CAP)],
                            comp_src.at[w, pl.ds(pl.multiple_of(hcur, GBLK), CAP)])
                        pltpu.sync_copy(
                            cdst.at[pl.ds(0, CAP)],
                            comp_dst.at[w, pl.ds(pl.multiple_of(hcur, GBLK), CAP)])

                def grp(gi, _):
                    for i in range(NBUF):
                        bi = gi * NBUF + i

                        @pl.when(bi < nb)
                        def _(i=i, bi=bi):
                            pltpu.async_copy(
                                table.at[csrc.at[pl.ds(bi * GBLK, GBLK)]],
                                rows[i], sems[i])
                    for i in range(NBUF):
                        bi = gi * NBUF + i

                        @pl.when(bi < nb)
                        def _(i=i, bi=bi):
                            pltpu.make_async_copy(
                                table.at[csrc.at[pl.ds(bi * GBLK, GBLK)]],
                                rows[i], sems[i]).wait()

                            if first:
                                for v in range(GBLK // LANES):
                                    dv = cdst[pl.ds(bi * GBLK + v * LANES,
                                                    LANES)]
                                    plsc.addupdate_scatter(
                                        cnt_acc, [dv - lo], ones_f)

                            def edge_body(e, _, i=i, bi=bi, acc=acc):
                                eidx = (jnp.zeros((LANES,), jnp.int32)
                                        + (bi * GBLK + e))
                                dstl = plsc.load_gather(cdst, [eidx]) - lo
                                for j in range(H // LANES):
                                    val = rows[i][e, pl.ds(j * LANES, LANES)]
                                    plsc.addupdate_scatter(
                                        acc, [dstl, colv[j]], val)
                                return 0

                            lax.fori_loop(0, GBLK, edge_body, 0)
                    return 0

                lax.fori_loop(0, ngr, grp, 0)
                hcur = hcur + (nb << 5)
                if final:
                    return jnp.int32(0), hcur
                rem_off = nb * GBLK
                for vv in range(2):
                    td = cdst[pl.ds(rem_off + vv * LANES, LANES)]
                    ts = csrc[pl.ds(rem_off + vv * LANES, LANES)]
                    cdst[pl.ds(vv * LANES, LANES)] = td
                    csrc[pl.ds(vv * LANES, LANES)] = ts
                return cur - (nb << 5), hcur

            if first:
                src_2d, dst_2d = src_hbm, dst_hbm
                nch = e_pads[t] // CHUNK_E

                def issue(ch, stg_s, stg_d, sem, src_2d=src_2d,
                          dst_2d=dst_2d):
                    pltpu.async_copy(
                        src_2d.at[pl.ds(ch * CHUNK_V, CHUNK_V)], stg_s, sem)
                    pltpu.async_copy(
                        dst_2d.at[pl.ds(ch * CHUNK_V, CHUNK_V)], stg_d, sem)

                def wait_stage(ch, stg_s, stg_d, sem, src_2d=src_2d,
                               dst_2d=dst_2d):
                    pltpu.make_async_copy(
                        src_2d.at[pl.ds(ch * CHUNK_V, CHUNK_V)],
                        stg_s, sem).wait()
                    pltpu.make_async_copy(
                        dst_2d.at[pl.ds(ch * CHUNK_V, CHUNK_V)],
                        stg_d, sem).wait()

                def scan_chunk(stg_s, stg_d, cur,
                               scan_v_factory=scan_v_factory):
                    sv_fn = scan_v_factory(
                        lambda v: stg_d[v, pl.ds(0, LANES)],
                        lambda v: stg_s[v, pl.ds(0, LANES)])
                    curv = jnp.zeros((LANES,), jnp.int32) + cur
                    curv = lax.fori_loop(0, CHUNK_V, sv_fn, curv)
                    return _scalar(curv)

                issue(0, srcA, dstA, semA)

                def pair_body(p, carry, nch=nch, flush=flush,
                              scan_chunk=scan_chunk, issue=issue,
                              wait_stage=wait_stage):
                    cur, hcur = carry
                    ch0 = 2 * p
                    wait_stage(ch0, srcA, dstA, semA)
                    issue(ch0 + 1, srcB, dstB, semB)
                    cur = scan_chunk(srcA, dstA, cur)
                    cur, hcur = flush(cur, hcur, False)
                    wait_stage(ch0 + 1, srcB, dstB, semB)

                    @pl.when(ch0 + 2 < nch)
                    def _():
                        issue(ch0 + 2, srcA, dstA, semA)

                    cur = scan_chunk(srcB, dstB, cur)
                    cur, hcur = flush(cur, hcur, False)
                    return cur, hcur

                cur, hcur = lax.fori_loop(
                    0, nch // 2, pair_body,
                    (jnp.int32(0), jnp.int32(0)))
                _, m_p = flush(cur, hcur, True)
                pltpu.sync_copy(dmybuf,
                                comp_dst.at[w, pl.ds(pl.multiple_of(m_p, GBLK), CHUNK_E)])
                mbuf[pl.ds(0, LANES)] = jnp.zeros((LANES,), jnp.int32) + m_p
                pltpu.sync_copy(mbuf, mcnt_out.at[w])
            else:
                pltpu.sync_copy(mcnt_hbm.at[w], mbuf)
                m_p = _scalar(mbuf[pl.ds(0, LANES)])
                nch_dyn = (m_p + CHUNK_E - 1) // CHUNK_E

                def chunk_body(ch, carry, comp_src=comp_src,
                               comp_dst=comp_dst, flush=flush,
                               scan_v_factory=scan_v_factory):
                    cur, hcur = carry
                    pltpu.sync_copy(
                        comp_src.at[w, pl.ds(pl.multiple_of(ch * CHUNK_E, CHUNK_E), CHUNK_E)], srcA)
                    pltpu.sync_copy(
                        comp_dst.at[w, pl.ds(pl.multiple_of(ch * CHUNK_E, CHUNK_E), CHUNK_E)], dstA)
                    sv_fn = scan_v_factory(
                        lambda v: dstA[pl.ds(v * LANES, LANES)],
                        lambda v: srcA[pl.ds(v * LANES, LANES)])
                    curv = jnp.zeros((LANES,), jnp.int32) + cur
                    curv = lax.fori_loop(0, CHUNK_V, sv_fn, curv)
                    cur = _scalar(curv)
                    cur, hcur = flush(cur, hcur, False)
                    return cur, hcur

                cur, _ = lax.fori_loop(0, nch_dyn, chunk_body,
                                       (jnp.int32(0), jnp.int32(0)))
                flush(cur, jnp.int32(0), True)

        for t in range(n_types):
            pltpu.sync_copy(accs[t].at[pl.ds(0, r_tile)], outs[t].at[w])
        if first:
            pltpu.sync_copy(cnt_acc.at[pl.ds(0, r_tile)], cnt_out.at[w])

    out_type = [jax.ShapeDtypeStruct((NW, r_tile, H), jnp.float32)
                for _ in range(n_types)]
    if first:
        out_type.append(jax.ShapeDtypeStruct((NW, r_tile), jnp.float32))
        for t in range(n_types):
            out_type += [jax.ShapeDtypeStruct((NW, cols[t]), jnp.int32),
                         jax.ShapeDtypeStruct((NW, cols[t]), jnp.int32),
                         jax.ShapeDtypeStruct((NW, LANES), jnp.int32)]
    scratch = [pltpu.VMEM((acc_rows, H), jnp.float32)
               for _ in range(n_types)]
    if first:
        scratch.append(pltpu.VMEM((acc_rows,), jnp.float32))
        scratch += [pltpu.VMEM((CHUNK_V, LANES), jnp.int32)
                    for _ in range(4)]
        scratch.append(pltpu.VMEM((CHUNK_E,), jnp.int32))
    else:
        scratch += [pltpu.VMEM((CHUNK_E,), jnp.int32) for _ in range(2)]
    scratch.append(pltpu.VMEM((LANES,), jnp.int32))
    scratch += [pltpu.VMEM((CAP,), jnp.int32) for _ in range(2)]
    scratch += [pltpu.VMEM((GBLK, H), jnp.float32) for _ in range(NBUF)]
    scratch += [pltpu.SemaphoreType.DMA for _ in range(2 + NBUF)]
    return pl.kernel(body, out_type=out_type, mesh=_mesh(),
                     scratch_types=scratch,
                     compiler_params=pltpu.CompilerParams(
                         use_tc_tiling_on_sc=False,
                         needs_layout_passes=False))


def _enc_body(x_ref, w_ref, b_ref, o_ref):
    o_ref[...] = jnp.dot(x_ref[...], w_ref[...],
                         preferred_element_type=jnp.float32) + b_ref[...]


def _encode(x, w, b, blk):
    n, f = x.shape
    return pl.pallas_call(
        _enc_body,
        grid=(n // blk,),
        in_specs=[pl.BlockSpec((blk, f), lambda i: (i, 0)),
                  pl.BlockSpec((f, H), lambda i: (0, 0)),
                  pl.BlockSpec((1, H), lambda i: (0, 0))],
        out_specs=pl.BlockSpec((blk, H), lambda i: (i, 0)),
        out_shape=jax.ShapeDtypeStruct((n, H), jnp.float32),
    )(x, w, b.reshape(1, H))


def _upd2_body(h_ref, a1_ref, a2_ref, cnt_ref, w1_ref, w2_ref, b_ref, o_ref):
    m = (jnp.dot(a1_ref[...], w1_ref[...], preferred_element_type=jnp.float32)
         + jnp.dot(a2_ref[...], w2_ref[...],
                   preferred_element_type=jnp.float32))
    m = m / jnp.maximum(cnt_ref[...], 1.0)
    o_ref[...] = h_ref[...] + jnp.maximum(m + b_ref[...], 0.0)


def _upd1_body(h_ref, a1_ref, cnt_ref, w1_ref, b_ref, o_ref):
    m = jnp.dot(a1_ref[...], w1_ref[...], preferred_element_type=jnp.float32)
    m = m / jnp.maximum(cnt_ref[...], 1.0)
    o_ref[...] = h_ref[...] + jnp.maximum(m + b_ref[...], 0.0)


def _update(h, aggs, cnt, ws, b, blk):
    n = h.shape[0]
    hspec = pl.BlockSpec((blk, H), lambda i: (i, 0))
    cnt_spec = pl.BlockSpec((blk, 1), lambda i: (i, 0))
    w_spec = pl.BlockSpec((H, H), lambda i: (0, 0))
    body = _upd2_body if len(aggs) == 2 else _upd1_body
    in_specs = ([hspec] + [hspec] * len(aggs) + [cnt_spec]
                + [w_spec] * len(ws)
                + [pl.BlockSpec((1, H), lambda i: (0, 0))])
    return pl.pallas_call(
        body,
        grid=(n // blk,),
        in_specs=in_specs,
        out_specs=hspec,
        out_shape=jax.ShapeDtypeStruct((n, H), jnp.float32),
    )(h, *aggs, cnt, *ws, b.reshape(1, H))


NC = 10000
NCASE = 50000
RT_C = 320
RT_S = 1568
NPC = NW * RT_C
NPS = NW * RT_S
NITER = 3


def kernel(x_county, x_case, ei_spatial, ei_genetic, ei_belongs,
           W_enc_county, b_enc_county, W_enc_case, b_enc_case,
           W_spatial, W_genetic, W_belongs, b_county, b_case):
    src_sp, dst_sp, ep_sp = _pad_edges(ei_spatial)
    src_ge, dst_ge, ep_ge = _pad_edges(ei_genetic)
    src_bl, dst_bl, ep_bl = _pad_edges(ei_belongs)

    xc = jnp.pad(x_county, ((0, NPC - NC), (0, 0)))
    xs = jnp.pad(x_case, ((0, NPS - NCASE), (0, 0)))
    hc = _encode(xc, W_enc_county, b_enc_county, 1024)
    hs = _encode(xs, W_enc_case, b_enc_case, 1024)

    county1 = _make_agg(2, (ep_sp, ep_bl), RT_C, "first")
    county2 = _make_agg(2, (ep_sp, ep_bl), RT_C, "rest")
    case1 = _make_agg(1, (ep_ge,), RT_S, "first")
    case2 = _make_agg(1, (ep_ge,), RT_S, "rest")

    cnt_c = cnt_s = None
    for it in range(NITER):
        if it == 0:
            (agg_sp, agg_bl, cnt_c,
             sp_cs, sp_cd, sp_m,
             bl_cs, bl_cd, bl_m) = county1(src_sp, dst_sp, hc,
                                           src_bl, dst_bl, hs)
            agg_ge, cnt_s, ge_cs, ge_cd, ge_m = case1(src_ge, dst_ge, hs)
            cnt_c = cnt_c.reshape(NPC, 1)
            cnt_s = cnt_s.reshape(NPS, 1)
        else:
            agg_sp, agg_bl = county2(sp_cs, sp_cd, sp_m, hc,
                                     bl_cs, bl_cd, bl_m, hs)
            (agg_ge,) = case2(ge_cs, ge_cd, ge_m, hs)
        hc = _update(hc, [agg_sp.reshape(NPC, H), agg_bl.reshape(NPC, H)],
                     cnt_c, [W_spatial, W_belongs], b_county, 1024)
        hs = _update(hs, [agg_ge.reshape(NPS, H)],
                     cnt_s, [W_genetic], b_case, 1024)
    return hc[:NC], hs[:NCASE]
